# baseline (speedup 1.0000x reference)
import functools
import math

import jax
import jax.numpy as jnp
from jax import lax
from jax.experimental import pallas as pl
from jax.experimental.pallas import tpu as pltpu
from jax.experimental.pallas import tpu_sc as plsc

D_MODEL = 1024
LANES = 16
NUM_CORES = 2
NUM_SUBCORES = 16
NUM_WORKERS = NUM_CORES * NUM_SUBCORES
CHUNK = 8
NBUF = 3


def _emb_body(xt_hbm, table_hbm, pe_hbm, out_hbm,
              idx_all, pe0, pe1, pe2, ga, gb, gc,
              psem0, psem1, psem2, gsem0, gsem1, gsem2,
              wsems0, wsems1, wsems2,
              *, batch, seq):
    scale = math.sqrt(D_MODEL)
    pos_per_w = seq // NUM_WORKERS
    n_groups = pos_per_w // CHUNK
    wid = lax.axis_index("s") * NUM_CORES + lax.axis_index("c")
    g0 = wid * n_groups

    pes = (pe0, pe1, pe2)
    psems = (psem0, psem1, psem2)
    bufs = (ga, gb, gc)
    gsems = (gsem0, gsem1, gsem2)
    wsems = (wsems0, wsems1, wsems2)

    def pe_slice(g):
        return pe_hbm.at[pl.ds((g0 + g) * CHUNK, CHUNK)]

    def issue_pe(s, g):
        pltpu.async_copy(pe_slice(g), pes[s], psems[s])

    def wait_pe(s):
        pltpu.make_async_copy(pe_slice(0), pes[s], psems[s]).wait()

    def issue_gather(s, g):
        pltpu.async_copy(table_hbm.at[idx_all.at[g]], bufs[s], gsems[s])

    def wait_gather(s):
        pltpu.make_async_copy(table_hbm.at[idx_all.at[0]], bufs[s],
                              gsems[s]).wait()

    def issue_write(s, b, g):
        pltpu.async_copy(bufs[s].at[pl.ds(b * CHUNK, CHUNK)],
                         out_hbm.at[pl.ds(b * seq + (g0 + g) * CHUNK, CHUNK)],
                         wsems[s].at[b])

    def wait_write(s, b):
        pltpu.make_async_copy(bufs[s].at[pl.ds(0, CHUNK)],
                              out_hbm.at[pl.ds(0, CHUNK)],
                              wsems[s].at[b]).wait()

    def compute_group(s):
        buf = bufs[s]
        pe_v = pes[s]

        def row_body(r, _):
            for j in range(D_MODEL // LANES):
                sl = pl.ds(j * LANES, LANES)
                pv = pe_v[r, sl]
                for b in range(batch):
                    buf[b * CHUNK + r, sl] = buf[b * CHUNK + r, sl] * scale + pv
            return 0

        lax.fori_loop(0, CHUNK, row_body, 0)

    def run_group(g, s, prefetch, guard_first=None):
        wait_pe(s)
        wait_gather(s)
        compute_group(s)
        for b in range(batch):
            issue_write(s, b, g)
        if prefetch:
            u = (s + 2) % NBUF

            def do_prefetch():
                for b in range(batch):
                    wait_write(u, b)
                issue_gather(u, g + 2)
                issue_pe(u, g + 2)

            if guard_first is None:
                do_prefetch()
            else:
                @pl.when(jnp.logical_not(guard_first))
                def _():
                    do_prefetch()

                @pl.when(guard_first)
                def _():
                    issue_gather(u, g + 2)
                    issue_pe(u, g + 2)

    pltpu.sync_copy(xt_hbm.at[pl.ds(g0, n_groups)], idx_all)
    issue_pe(0, 0)
    issue_gather(0, 0)
    issue_pe(1, 1)
    issue_gather(1, 1)

    def outer(i, _):
        g = 3 * i
        run_group(g, 0, prefetch=True, guard_first=(i == 0))
        run_group(g + 1, 1, prefetch=True)
        run_group(g + 2, 2, prefetch=True)
        return 0

    lax.fori_loop(0, (n_groups - 2) // 3, outer, 0)
    run_group(n_groups - 2, 0, prefetch=False)
    run_group(n_groups - 1, 1, prefetch=False)
    for s in range(NBUF):
        for b in range(batch):
            wait_write(s, b)


def kernel(x, table, pe):
    batch, seq = x.shape
    xt = (x.T.reshape(seq // CHUNK, CHUNK, batch)
          .transpose(0, 2, 1).reshape(seq // CHUNK, batch * CHUNK))
    pe2d = pe[0, :seq, :]

    mesh = plsc.VectorSubcoreMesh(core_axis_name="c", subcore_axis_name="s")
    k = pl.kernel(
        functools.partial(_emb_body, batch=batch, seq=seq),
        mesh=mesh,
        out_type=jax.ShapeDtypeStruct((batch * seq, D_MODEL), jnp.float32),
        scratch_types=[
            pltpu.VMEM((seq // NUM_WORKERS // CHUNK, batch * CHUNK),
                       jnp.int32),
            pltpu.VMEM((CHUNK, D_MODEL), jnp.float32),
            pltpu.VMEM((CHUNK, D_MODEL), jnp.float32),
            pltpu.VMEM((CHUNK, D_MODEL), jnp.float32),
            pltpu.VMEM((batch * CHUNK, D_MODEL), jnp.float32),
            pltpu.VMEM((batch * CHUNK, D_MODEL), jnp.float32),
            pltpu.VMEM((batch * CHUNK, D_MODEL), jnp.float32),
            pltpu.SemaphoreType.DMA,
            pltpu.SemaphoreType.DMA,
            pltpu.SemaphoreType.DMA,
            pltpu.SemaphoreType.DMA,
            pltpu.SemaphoreType.DMA,
            pltpu.SemaphoreType.DMA,
            pltpu.SemaphoreType.DMA((4,)),
            pltpu.SemaphoreType.DMA((4,)),
            pltpu.SemaphoreType.DMA((4,)),
        ],
    )
    out = k(xt, table, pe2d)
    return out.reshape(batch, seq, D_MODEL)


# speedup vs baseline: 1.0412x; 1.0412x over previous
import functools
import math

import jax
import jax.numpy as jnp
from jax import lax
from jax.experimental import pallas as pl
from jax.experimental.pallas import tpu as pltpu
from jax.experimental.pallas import tpu_sc as plsc

D_MODEL = 1024
LANES = 16
NUM_CORES = 2
NUM_SUBCORES = 16
NUM_WORKERS = NUM_CORES * NUM_SUBCORES
CHUNK = 8


def _emb_body(xt_hbm, table_hbm, pe_hbm, out_hbm,
              idx_all, pe0, pe1, ga, gb,
              psem0, psem1, gsem0, gsem1, wsems0, wsems1,
              *, batch, seq):
    scale = math.sqrt(D_MODEL)
    pos_per_w = seq // NUM_WORKERS
    n_groups = pos_per_w // CHUNK
    wid = lax.axis_index("s") * NUM_CORES + lax.axis_index("c")
    g0 = wid * n_groups

    pes = (pe0, pe1)
    psems = (psem0, psem1)
    bufs = (ga, gb)
    gsems = (gsem0, gsem1)
    wsems = (wsems0, wsems1)

    def pe_slice(g):
        return pe_hbm.at[pl.ds((g0 + g) * CHUNK, CHUNK)]

    def issue_pe(s, g):
        pltpu.async_copy(pe_slice(g), pes[s], psems[s])

    def wait_pe(s):
        pltpu.make_async_copy(pe_slice(0), pes[s], psems[s]).wait()

    def issue_gather(s, g):
        pltpu.async_copy(table_hbm.at[idx_all.at[g]], bufs[s], gsems[s])

    def wait_gather(s):
        pltpu.make_async_copy(table_hbm.at[idx_all.at[0]], bufs[s],
                              gsems[s]).wait()

    def issue_write(s, b, g):
        pltpu.async_copy(bufs[s].at[pl.ds(b * CHUNK, CHUNK)],
                         out_hbm.at[pl.ds(b * seq + (g0 + g) * CHUNK, CHUNK)],
                         wsems[s].at[b])

    def wait_write(s, b):
        pltpu.make_async_copy(bufs[s].at[pl.ds(0, CHUNK)],
                              out_hbm.at[pl.ds(0, CHUNK)],
                              wsems[s].at[b]).wait()

    def compute_group(s):
        buf = bufs[s]
        pe_v = pes[s]

        def row_body(r, _):
            for j in range(D_MODEL // LANES):
                sl = pl.ds(j * LANES, LANES)
                pv = pe_v[r, sl]
                for b in range(batch):
                    buf[b * CHUNK + r, sl] = buf[b * CHUNK + r, sl] * scale + pv
            return 0

        lax.fori_loop(0, CHUNK, row_body, 0)

    def run_group(g, s, first, last):
        t = 1 - s

        wait_gather(s)
        wait_pe(s)

        @pl.when(jnp.logical_not(last))
        def _():
            @pl.when(jnp.logical_not(first))
            def _():
                for b in range(batch):
                    wait_write(t, b)
            issue_gather(t, g + 1)
            issue_pe(t, g + 1)

        compute_group(s)
        for b in range(batch):
            issue_write(s, b, g)

    pltpu.sync_copy(xt_hbm.at[pl.ds(g0, n_groups)], idx_all)
    issue_pe(0, 0)
    issue_gather(0, 0)

    def outer(i, _):
        g = 2 * i
        run_group(g, 0, first=(g == 0), last=jnp.bool_(False))
        run_group(g + 1, 1, first=jnp.bool_(False),
                  last=(g + 1 == n_groups - 1))
        return 0

    lax.fori_loop(0, n_groups // 2, outer, 0)
    for b in range(batch):
        wait_write(0, b)
        wait_write(1, b)


def kernel(x, table, pe):
    batch, seq = x.shape
    xt = (x.T.reshape(seq // CHUNK, CHUNK, batch)
          .transpose(0, 2, 1).reshape(seq // CHUNK, batch * CHUNK))
    pe2d = pe[0, :seq, :]

    mesh = plsc.VectorSubcoreMesh(core_axis_name="c", subcore_axis_name="s")
    k = pl.kernel(
        functools.partial(_emb_body, batch=batch, seq=seq),
        mesh=mesh,
        out_type=jax.ShapeDtypeStruct((batch * seq, D_MODEL), jnp.float32),
        scratch_types=[
            pltpu.VMEM((seq // NUM_WORKERS // CHUNK, batch * CHUNK),
                       jnp.int32),
            pltpu.VMEM((CHUNK, D_MODEL), jnp.float32),
            pltpu.VMEM((CHUNK, D_MODEL), jnp.float32),
            pltpu.VMEM((batch * CHUNK, D_MODEL), jnp.float32),
            pltpu.VMEM((batch * CHUNK, D_MODEL), jnp.float32),
            pltpu.SemaphoreType.DMA,
            pltpu.SemaphoreType.DMA,
            pltpu.SemaphoreType.DMA,
            pltpu.SemaphoreType.DMA,
            pltpu.SemaphoreType.DMA((4,)),
            pltpu.SemaphoreType.DMA((4,)),
        ],
    )
    out = k(xt, table, pe2d)
    return out.reshape(batch, seq, D_MODEL)
